# baseline (speedup 1.0000x reference)
import functools

import jax
import jax.numpy as jnp
from jax import lax
from jax.experimental import pallas as pl
from jax.experimental.pallas import tpu as pltpu
from jax.experimental.pallas import tpu_sc as plsc

NUM_CORES = 2
NUM_SUBCORES = 16
NUM_WORKERS = NUM_CORES * NUM_SUBCORES

CH = 512
NBUF = 2


def _make_gather(S, T, V, D):
    B = S * T
    n_chunks_total = B // CH
    k_per_w = n_chunks_total // NUM_WORKERS
    b_per_w = k_per_w * CH
    st_per_chunk = CH // 128
    chunks_per_t = S // CH
    DT, DS = D // 8, 8
    assert b_per_w * NUM_WORKERS == B and CH * chunks_per_t == S
    assert k_per_w % NBUF == 0 and k_per_w >= 3 * NBUF
    n_steps = k_per_w // NBUF

    mesh = plsc.VectorSubcoreMesh(core_axis_name="c", subcore_axis_name="s")

    scratch = (
        [pltpu.VMEM((b_per_w,), jnp.int32)]
        + [pltpu.VMEM((CH, D), jnp.float32) for _ in range(NBUF)]
        + [pltpu.VMEM((DT, st_per_chunk, DS, 128), jnp.float32) for _ in range(NBUF)]
        + [pltpu.SemaphoreType.DMA for _ in range(2 * NBUF)]
    )

    @functools.partial(
        pl.kernel,
        out_type=jax.ShapeDtypeStruct((T, DT, S // 128, DS, 128), jnp.float32),
        mesh=mesh,
        scratch_types=scratch,
        compiler_params=pltpu.CompilerParams(
            use_tc_tiling_on_sc=False, needs_layout_passes=False
        ),
    )
    def gather_kernel(idx_hbm, table_hbm, out_hbm, idx_v, *bufs):
        rows = bufs[:NBUF]
        obuf = bufs[NBUF : 2 * NBUF]
        gsem = bufs[2 * NBUF : 3 * NBUF]
        ssem = bufs[3 * NBUF :]
        wid = lax.axis_index("s") * NUM_CORES + lax.axis_index("c")
        k0 = wid * k_per_w
        pltpu.sync_copy(idx_hbm.at[pl.ds(k0 * CH, b_per_w)], idx_v)

        iotav = lax.iota(jnp.int32, 16)
        dsplat = [jnp.full((16,), d, jnp.int32) for d in range(D)]

        def start_gather(c, b):
            pltpu.async_copy(
                table_hbm.at[idx_v.at[pl.ds(c * CH, CH)]], rows[b], gsem[b]
            )

        def wait_gather(b):
            pltpu.make_async_copy(
                table_hbm.at[idx_v.at[pl.ds(0, CH)]], rows[b], gsem[b]
            ).wait()

        def out_slice(c):
            k = k0 + c
            t = k // chunks_per_t
            st0 = (k % chunks_per_t) * st_per_chunk
            return out_hbm.at[
                t, slice(None), pl.ds(st0, st_per_chunk), slice(None), slice(None)
            ]

        def start_store(c, b):
            pltpu.async_copy(obuf[b], out_slice(c), ssem[b])

        def wait_store(b):
            pltpu.make_async_copy(obuf[b], out_slice(0), ssem[b]).wait()

        def shuffle(b):
            rv = rows[b]
            ob = obuf[b]

            def qbody(q, carry):
                stl = q // 8
                sl0 = (q % 8) * 16
                jvec = q * 16 + iotav
                for d in range(D):
                    v = plsc.load_gather(rv, [jvec, dsplat[d]])
                    ob[d // 8, stl, d % 8, pl.ds(sl0, 16)] = v
                return carry

            lax.fori_loop(0, (CH // 16), qbody, 0)

        for b in range(NBUF):
            start_gather(b, b)

        def step_body(step, carry):
            for b in range(NBUF):
                c = step * NBUF + b
                wait_gather(b)
                shuffle(b)

                @pl.when(step > 0)
                def _():
                    wait_store(b)

                start_store(c, b)
                start_gather(c + NBUF, b)
            return carry

        lax.fori_loop(0, n_steps - 1, step_body, 0)

        for b in range(NBUF):
            c = (n_steps - 1) * NBUF + b
            wait_gather(b)
            shuffle(b)
            wait_store(b)
            start_store(c, b)
        for b in range(NBUF):
            wait_store(b)

    return gather_kernel


def kernel(phonemes, table):
    S, T = phonemes.shape
    V, D = table.shape
    idx_flat = jnp.transpose(phonemes).reshape(-1).astype(jnp.int32)
    out5 = _make_gather(S, T, V, D)(idx_flat, table)
    x = out5.transpose(0, 1, 3, 2, 4).reshape(T, D, S)
    return x.transpose(2, 0, 1)
